# baseline (speedup 1.0000x reference)
import functools

import jax
import jax.numpy as jnp
from jax import lax
from jax.experimental import pallas as pl
from jax.experimental.pallas import tpu as pltpu
from jax.experimental.pallas import tpu_sc as plsc

N = 10000
E = 320000
D = 128

NC = 2
NS = 16
NW = NC * NS
L = 16

OWN = 320
NPAD = OWN * NW
G = 128
SUBCAP = 896
CAP = SUBCAP * L
CHUNK = 6400
NCH = E // CHUNK

_mesh = plsc.VectorSubcoreMesh(core_axis_name="c", subcore_axis_name="s")


def _wid():
    return lax.axis_index("c") * NS + lax.axis_index("s")


@functools.partial(
    pl.kernel,
    out_type=(
        jax.ShapeDtypeStruct((NW, CAP), jnp.int32),
        jax.ShapeDtypeStruct((NW, CAP), jnp.int32),
        jax.ShapeDtypeStruct((NW, L), jnp.int32),
    ),
    mesh=_mesh,
    compiler_params=pltpu.CompilerParams(needs_layout_passes=False),
    scratch_types=[
        pltpu.VMEM((2, CHUNK), jnp.int32),
        pltpu.VMEM((2, CHUNK), jnp.int32),
        pltpu.VMEM((CAP,), jnp.int32),
        pltpu.VMEM((CAP,), jnp.int32),
        pltpu.VMEM((CAP,), jnp.int32),
        pltpu.VMEM((CAP,), jnp.int32),
        pltpu.VMEM((L,), jnp.int32),
        pltpu.SemaphoreType.DMA,
        pltpu.SemaphoreType.DMA,
    ],
)
def _compact(src_hbm, dst_hbm, srcl_hbm, offl_hbm, cnt_hbm,
             srcc_v, dstc_v, srcs_v, offs_v, srcl_v, offl_v, cnt_v,
             sem0, sem1):
    wid = _wid()
    lo = wid * OWN
    hi = lo + OWN
    lane_base = lax.iota(jnp.int32, L) * SUBCAP
    sems = (sem0, sem1)

    def start_chunk(ci, b):
        base = ci * CHUNK
        pltpu.make_async_copy(src_hbm.at[pl.ds(base, CHUNK)],
                              srcc_v.at[b], sems[b]).start()
        pltpu.make_async_copy(dst_hbm.at[pl.ds(base, CHUNK)],
                              dstc_v.at[b], sems[b]).start()

    def wait_chunk(b):
        pltpu.make_async_copy(src_hbm.at[pl.ds(0, CHUNK)],
                              srcc_v.at[b], sems[b]).wait()
        pltpu.make_async_copy(dst_hbm.at[pl.ds(0, CHUNK)],
                              dstc_v.at[b], sems[b]).wait()

    start_chunk(0, 0)

    def outer(gg, ptrs):
        for b in range(2):
            ci = gg * 2 + b

            @pl.when(ci + 1 < NCH)
            def _():
                start_chunk(ci + 1, 1 - b)

            wait_chunk(b)

            def vec_body(i, p):
                dv = dstc_v[b, pl.ds(i * L, L)]
                sv = srcc_v[b, pl.ds(i * L, L)]
                m = (dv >= lo) & (dv < hi)
                pos = lane_base + p
                plsc.store_scatter(srcs_v, [pos], sv, mask=m)
                plsc.store_scatter(offs_v, [pos], dv - lo, mask=m)
                return jnp.minimum(p + m.astype(jnp.int32), SUBCAP - L)

            ptrs = lax.fori_loop(0, CHUNK // L, vec_body, ptrs)
        return ptrs

    ptrs = lax.fori_loop(0, NCH // 2, outer, jnp.zeros((L,), jnp.int32))

    off = jnp.int32(0)
    for l in range(L):
        c_l = ptrs[l]
        src_base = l * SUBCAP

        def cp(i, _, off=off, src_base=src_base):
            srcl_v[pl.ds(off + i * L, L)] = srcs_v[pl.ds(src_base + i * L, L)]
            offl_v[pl.ds(off + i * L, L)] = offs_v[pl.ds(src_base + i * L, L)]
            return 0

        lax.fori_loop(0, (c_l + (L - 1)) // L, cp, 0)
        off = off + c_l

    pad_s = jnp.zeros((L,), jnp.int32)
    pad_o = jnp.full((L,), OWN, jnp.int32)

    def pad_body(j, _):
        srcl_v[pl.ds(off + j * L, L)] = pad_s
        offl_v[pl.ds(off + j * L, L)] = pad_o
        return 0

    lax.fori_loop(0, G // L, pad_body, 0)

    cnt_v[...] = jnp.full((L,), off, jnp.int32)
    pltpu.sync_copy(srcl_v, srcl_hbm.at[wid])
    pltpu.sync_copy(offl_v, offl_hbm.at[wid])
    pltpu.sync_copy(cnt_v, cnt_hbm.at[wid])


@functools.partial(
    pl.kernel,
    out_type=jax.ShapeDtypeStruct((NPAD, D), jnp.float32),
    mesh=_mesh,
    scratch_types=[
        pltpu.VMEM((2, G), jnp.int32),
        pltpu.VMEM((2, G), jnp.int32),
        pltpu.VMEM((L,), jnp.int32),
        pltpu.VMEM((OWN + 8, D), jnp.float32),
        pltpu.VMEM((OWN + 8, D), jnp.float32),
        pltpu.VMEM((2, G, D), jnp.float32),
        pltpu.SemaphoreType.DMA,
        pltpu.SemaphoreType.DMA,
        pltpu.SemaphoreType.DMA,
        pltpu.SemaphoreType.DMA,
    ],
)
def _segmax(m_hbm, srcl_hbm, offl_hbm, cnt_hbm, agg_hbm,
            idx_v, off_v, cnt_v, accA, accB, rows_v,
            semi0, semi1, semg0, semg1):
    wid = _wid()
    pltpu.sync_copy(cnt_hbm.at[wid], cnt_v)

    zero = jnp.zeros((L,), jnp.float32)

    def zrow(r, _):
        for c in range(D // L):
            accA[r, pl.ds(c * L, L)] = zero
            accB[r, pl.ds(c * L, L)] = zero
        return 0

    lax.fori_loop(0, OWN, zrow, 0)

    cnt = cnt_v[...][0]
    ngroups = (cnt + (G - 1)) // G
    semi = (semi0, semi1)
    semg = (semg0, semg1)

    def start_io(g, b):
        pltpu.make_async_copy(srcl_hbm.at[wid, pl.ds(g * G, G)],
                              idx_v.at[b], semi[b]).start()
        pltpu.make_async_copy(offl_hbm.at[wid, pl.ds(g * G, G)],
                              off_v.at[b], semi[b]).start()

    def wait_io(b):
        pltpu.make_async_copy(srcl_hbm.at[wid, pl.ds(0, G)],
                              idx_v.at[b], semi[b]).wait()
        pltpu.make_async_copy(offl_hbm.at[wid, pl.ds(0, G)],
                              off_v.at[b], semi[b]).wait()

    def start_gather(b):
        pltpu.make_async_copy(m_hbm.at[idx_v.at[b]],
                              rows_v.at[b], semg[b]).start()

    def wait_gather(b):
        pltpu.make_async_copy(m_hbm.at[idx_v.at[b]],
                              rows_v.at[b], semg[b]).wait()

    @pl.when(ngroups > 0)
    def _():
        start_io(0, 0)

    @pl.when(ngroups > 1)
    def _():
        start_io(1, 1)

    @pl.when(ngroups > 0)
    def _():
        wait_io(0)
        start_gather(0)

    def outer(gg, _):
        for b in range(2):
            g = gg * 2 + b

            @pl.when(g < ngroups)
            def _():
                wait_gather(b)

                @pl.when(g + 1 < ngroups)
                def _():
                    wait_io(1 - b)
                    start_gather(1 - b)

                def blk_body(k, _):
                    ovec = off_v[b, pl.ds(k * L, L)]
                    for j in range(L):
                        d = ovec[j]
                        acc = accA if j % 2 == 0 else accB
                        e = k * L + j
                        for c in range(D // L):
                            sl = pl.ds(c * L, L)
                            acc[d, sl] = jnp.maximum(
                                acc[d, sl], rows_v[b, e, sl])
                    return 0

                lax.fori_loop(0, G // L, blk_body, 0)

                @pl.when(g + 2 < ngroups)
                def _():
                    start_io(g + 2, b)
        return 0

    lax.fori_loop(0, (ngroups + 1) // 2, outer, 0)

    def mrow(r, _):
        for c in range(D // L):
            sl = pl.ds(c * L, L)
            accA[r, sl] = jnp.maximum(accA[r, sl], accB[r, sl])
        return 0

    lax.fori_loop(0, OWN, mrow, 0)
    pltpu.sync_copy(accA.at[pl.ds(0, OWN)], agg_hbm.at[pl.ds(wid * OWN, OWN)])


BLK = 2000
NB = N // BLK


def _mm_relu_body(x_ref, w_ref, b_ref, o_ref):
    o_ref[...] = jnp.maximum(
        jnp.dot(x_ref[...], w_ref[...], preferred_element_type=jnp.float32)
        + b_ref[...], 0.0)


def _mm_relu(x, w, b):
    return pl.pallas_call(
        _mm_relu_body,
        grid=(NB,),
        in_specs=[
            pl.BlockSpec((BLK, D), lambda i: (i, 0)),
            pl.BlockSpec((D, D), lambda i: (0, 0)),
            pl.BlockSpec((1, D), lambda i: (0, 0)),
        ],
        out_specs=pl.BlockSpec((BLK, D), lambda i: (i, 0)),
        out_shape=jax.ShapeDtypeStruct((N, D), jnp.float32),
    )(x, w, b.reshape(1, D))


def _fc_pre_body(x_ref, a_ref, wa_ref, wb_ref, b_ref, pre_ref, st_ref):
    pre = (jnp.dot(x_ref[...], wa_ref[...], preferred_element_type=jnp.float32)
           + jnp.dot(a_ref[...], wb_ref[...], preferred_element_type=jnp.float32)
           + b_ref[...])
    pre = jnp.maximum(pre, 0.0)
    pre_ref[...] = pre
    st_ref[0, 0:1, :] = jnp.sum(pre, axis=0, keepdims=True)
    st_ref[0, 1:2, :] = jnp.sum(pre * pre, axis=0, keepdims=True)


def _fc_pre(x, a, wa, wb, b):
    return pl.pallas_call(
        _fc_pre_body,
        grid=(NB,),
        in_specs=[
            pl.BlockSpec((BLK, D), lambda i: (i, 0)),
            pl.BlockSpec((BLK, D), lambda i: (i, 0)),
            pl.BlockSpec((D, D), lambda i: (0, 0)),
            pl.BlockSpec((D, D), lambda i: (0, 0)),
            pl.BlockSpec((1, D), lambda i: (0, 0)),
        ],
        out_specs=[
            pl.BlockSpec((BLK, D), lambda i: (i, 0)),
            pl.BlockSpec((1, 2, D), lambda i: (i, 0, 0)),
        ],
        out_shape=[
            jax.ShapeDtypeStruct((N, D), jnp.float32),
            jax.ShapeDtypeStruct((NB, 2, D), jnp.float32),
        ],
    )(x, a, wa, wb, b.reshape(1, D))


def _bn_body(pre_ref, st_ref, g_ref, be_ref, w2_ref, b2_ref, out1_ref, m2_ref):
    s = jnp.sum(st_ref[...], axis=0)
    mean = s[0:1, :] / N
    var = s[1:2, :] / N - mean * mean
    inv = g_ref[...] * lax.rsqrt(var + 1e-5)
    x = (pre_ref[...] - mean) * inv + be_ref[...]
    nrm = jnp.sqrt(jnp.sum(x * x, axis=1, keepdims=True))
    out1 = x / (nrm + 1e-6)
    out1_ref[...] = out1
    m2_ref[...] = jnp.maximum(
        jnp.dot(out1, w2_ref[...], preferred_element_type=jnp.float32)
        + b2_ref[...], 0.0)


def _bn_m2(pre, stats, gamma, beta, w2, b2):
    return pl.pallas_call(
        _bn_body,
        grid=(NB,),
        in_specs=[
            pl.BlockSpec((BLK, D), lambda i: (i, 0)),
            pl.BlockSpec((NB, 2, D), lambda i: (0, 0, 0)),
            pl.BlockSpec((1, D), lambda i: (0, 0)),
            pl.BlockSpec((1, D), lambda i: (0, 0)),
            pl.BlockSpec((D, D), lambda i: (0, 0)),
            pl.BlockSpec((1, D), lambda i: (0, 0)),
        ],
        out_specs=[
            pl.BlockSpec((BLK, D), lambda i: (i, 0)),
            pl.BlockSpec((BLK, D), lambda i: (i, 0)),
        ],
        out_shape=[
            jax.ShapeDtypeStruct((N, D), jnp.float32),
            jax.ShapeDtypeStruct((N, D), jnp.float32),
        ],
    )(pre, stats, gamma.reshape(1, D), beta.reshape(1, D), w2, b2.reshape(1, D))


def _fc_final_body(x_ref, a_ref, wa_ref, wb_ref, b_ref, o_ref):
    o_ref[...] = (
        jnp.dot(x_ref[...], wa_ref[...], preferred_element_type=jnp.float32)
        + jnp.dot(a_ref[...], wb_ref[...], preferred_element_type=jnp.float32)
        + b_ref[...])


def _fc_final(x, a, wa, wb, b):
    return pl.pallas_call(
        _fc_final_body,
        grid=(NB,),
        in_specs=[
            pl.BlockSpec((BLK, D), lambda i: (i, 0)),
            pl.BlockSpec((BLK, D), lambda i: (i, 0)),
            pl.BlockSpec((D, D), lambda i: (0, 0)),
            pl.BlockSpec((D, D), lambda i: (0, 0)),
            pl.BlockSpec((1, D), lambda i: (0, 0)),
        ],
        out_specs=pl.BlockSpec((BLK, D), lambda i: (i, 0)),
        out_shape=jax.ShapeDtypeStruct((N, D), jnp.float32),
    )(x, a, wa, wb, b.reshape(1, D))


def kernel(features, edge_index, W_agg1, b_agg1, W_fc1, b_fc1, gamma, beta,
           W_agg2, b_agg2, W_fc2, b_fc2):
    src = edge_index[0]
    dst = edge_index[1]

    srcl, offl, cnts = _compact(src, dst)
    m1 = _mm_relu(features, W_agg1, b_agg1)
    agg1 = _segmax(m1, srcl, offl, cnts)[:N]
    pre, stats = _fc_pre(features, agg1, W_fc1[:D], W_fc1[D:], b_fc1)
    out1, m2 = _bn_m2(pre, stats, gamma, beta, W_agg2, b_agg2)
    agg2 = _segmax(m2, srcl, offl, cnts)[:N]
    return _fc_final(out1, agg2, W_fc2[:D], W_fc2[D:], b_fc2)


# speedup vs baseline: 1.6804x; 1.6804x over previous
import functools

import jax
import jax.numpy as jnp
from jax import lax
from jax.experimental import pallas as pl
from jax.experimental.pallas import tpu as pltpu
from jax.experimental.pallas import tpu_sc as plsc

N = 10000
E = 320000
D = 128

NC = 2
NS = 16
NW = NC * NS
L = 16

OWN = 320
NPAD = OWN * NW
G = 128
SUBCAP = 896
CAP = SUBCAP * L
CHUNK = 6400
NCH = E // CHUNK

_mesh = plsc.VectorSubcoreMesh(core_axis_name="c", subcore_axis_name="s")


def _wid():
    return lax.axis_index("c") * NS + lax.axis_index("s")


@functools.partial(
    pl.kernel,
    out_type=(
        jax.ShapeDtypeStruct((NW, CAP), jnp.int32),
        jax.ShapeDtypeStruct((NW, CAP), jnp.int32),
        jax.ShapeDtypeStruct((NW, L), jnp.int32),
    ),
    mesh=_mesh,
    compiler_params=pltpu.CompilerParams(needs_layout_passes=False),
    scratch_types=[
        pltpu.VMEM((2, CHUNK), jnp.int32),
        pltpu.VMEM((2, CHUNK), jnp.int32),
        pltpu.VMEM((CAP,), jnp.int32),
        pltpu.VMEM((CAP,), jnp.int32),
        pltpu.VMEM((CAP,), jnp.int32),
        pltpu.VMEM((CAP,), jnp.int32),
        pltpu.VMEM((L,), jnp.int32),
        pltpu.SemaphoreType.DMA,
        pltpu.SemaphoreType.DMA,
    ],
)
def _compact(src_hbm, dst_hbm, srcl_hbm, offl_hbm, cnt_hbm,
             srcc_v, dstc_v, srcs_v, offs_v, srcl_v, offl_v, cnt_v,
             sem0, sem1):
    wid = _wid()
    lo = wid * OWN
    hi = lo + OWN
    lane_base = lax.iota(jnp.int32, L) * SUBCAP
    sems = (sem0, sem1)

    def start_chunk(ci, b):
        base = ci * CHUNK
        pltpu.make_async_copy(src_hbm.at[pl.ds(base, CHUNK)],
                              srcc_v.at[b], sems[b]).start()
        pltpu.make_async_copy(dst_hbm.at[pl.ds(base, CHUNK)],
                              dstc_v.at[b], sems[b]).start()

    def wait_chunk(b):
        pltpu.make_async_copy(src_hbm.at[pl.ds(0, CHUNK)],
                              srcc_v.at[b], sems[b]).wait()
        pltpu.make_async_copy(dst_hbm.at[pl.ds(0, CHUNK)],
                              dstc_v.at[b], sems[b]).wait()

    start_chunk(0, 0)

    def outer(gg, ptrs):
        for b in range(2):
            ci = gg * 2 + b

            @pl.when(ci + 1 < NCH)
            def _():
                start_chunk(ci + 1, 1 - b)

            wait_chunk(b)

            def vec_body(i, p):
                dv = dstc_v[b, pl.ds(i * L, L)]
                sv = srcc_v[b, pl.ds(i * L, L)]
                m = (dv >= lo) & (dv < hi)
                pos = lane_base + p
                plsc.store_scatter(srcs_v, [pos], sv, mask=m)
                plsc.store_scatter(offs_v, [pos], dv - lo, mask=m)
                return jnp.minimum(p + m.astype(jnp.int32), SUBCAP - L)

            ptrs = lax.fori_loop(0, CHUNK // L, vec_body, ptrs)
        return ptrs

    ptrs = lax.fori_loop(0, NCH // 2, outer, jnp.zeros((L,), jnp.int32))

    off = jnp.int32(0)
    for l in range(L):
        c_l = ptrs[l]
        src_base = l * SUBCAP

        def cp(i, _, off=off, src_base=src_base):
            srcl_v[pl.ds(off + i * L, L)] = srcs_v[pl.ds(src_base + i * L, L)]
            offl_v[pl.ds(off + i * L, L)] = offs_v[pl.ds(src_base + i * L, L)]
            return 0

        lax.fori_loop(0, (c_l + (L - 1)) // L, cp, 0)
        off = off + c_l

    pad_s = jnp.zeros((L,), jnp.int32)
    pad_o = jnp.full((L,), OWN, jnp.int32)

    def pad_body(j, _):
        srcl_v[pl.ds(off + j * L, L)] = pad_s
        offl_v[pl.ds(off + j * L, L)] = pad_o
        return 0

    lax.fori_loop(0, G // L, pad_body, 0)

    cnt_v[...] = jnp.full((L,), off, jnp.int32)
    pltpu.sync_copy(srcl_v, srcl_hbm.at[wid])
    pltpu.sync_copy(offl_v, offl_hbm.at[wid])
    pltpu.sync_copy(cnt_v, cnt_hbm.at[wid])


@functools.partial(
    pl.kernel,
    out_type=jax.ShapeDtypeStruct((NPAD, D), jnp.float32),
    mesh=_mesh,
    scratch_types=[
        pltpu.VMEM((CAP,), jnp.int32),
        pltpu.VMEM((CAP,), jnp.int32),
        pltpu.VMEM((L,), jnp.int32),
        pltpu.VMEM((OWN + 8, D), jnp.float32),
        pltpu.VMEM((2, G, D), jnp.float32),
        pltpu.SemaphoreType.DMA,
        pltpu.SemaphoreType.DMA,
    ],
)
def _segmax(m_hbm, srcl_hbm, offl_hbm, cnt_hbm, agg_hbm,
            srcl_v, offl_v, cnt_v, acc_v, rows_v, sem0, sem1):
    wid = _wid()
    pltpu.sync_copy(srcl_hbm.at[wid], srcl_v)
    pltpu.sync_copy(offl_hbm.at[wid], offl_v)
    pltpu.sync_copy(cnt_hbm.at[wid], cnt_v)

    zero = jnp.zeros((L,), jnp.float32)

    def zrow(r, _):
        for c in range(D // L):
            acc_v[r, pl.ds(c * L, L)] = zero
        return 0

    lax.fori_loop(0, OWN, zrow, 0)

    cnt = cnt_v[...][0]
    ngroups = (cnt + (G - 1)) // G
    sems = (sem0, sem1)

    def start_gather(g, b):
        pltpu.make_async_copy(m_hbm.at[srcl_v.at[pl.ds(g * G, G)]],
                              rows_v.at[b], sems[b]).start()

    def wait_gather(b):
        pltpu.make_async_copy(m_hbm.at[srcl_v.at[pl.ds(0, G)]],
                              rows_v.at[b], sems[b]).wait()

    @pl.when(ngroups > 0)
    def _():
        start_gather(0, 0)

    NC_ = D // L

    def outer(gg, _):
        for b in range(2):
            g = gg * 2 + b

            @pl.when(g < ngroups)
            def _():
                @pl.when(g + 1 < ngroups)
                def _():
                    start_gather(g + 1, 1 - b)

                wait_gather(b)

                def blk_body(k, _):
                    ovec = offl_v[pl.ds(g * G + k * L, L)]
                    dsts = [ovec[j] for j in range(L)]
                    for j in range(L):
                        d = dsts[j]
                        e = k * L + j
                        rvals = [rows_v[b, e, pl.ds(c * L, L)]
                                 for c in range(NC_)]
                        avals = [acc_v[d, pl.ds(c * L, L)]
                                 for c in range(NC_)]
                        for c in range(NC_):
                            acc_v[d, pl.ds(c * L, L)] = jnp.maximum(
                                avals[c], rvals[c])
                    return 0

                lax.fori_loop(0, G // L, blk_body, 0)
        return 0

    lax.fori_loop(0, (ngroups + 1) // 2, outer, 0)
    pltpu.sync_copy(acc_v.at[pl.ds(0, OWN)], agg_hbm.at[pl.ds(wid * OWN, OWN)])


BLK = 2000
NB = N // BLK


def _mm_relu_body(x_ref, w_ref, b_ref, o_ref):
    o_ref[...] = jnp.maximum(
        jnp.dot(x_ref[...], w_ref[...], preferred_element_type=jnp.float32)
        + b_ref[...], 0.0)


def _mm_relu(x, w, b):
    return pl.pallas_call(
        _mm_relu_body,
        grid=(NB,),
        in_specs=[
            pl.BlockSpec((BLK, D), lambda i: (i, 0)),
            pl.BlockSpec((D, D), lambda i: (0, 0)),
            pl.BlockSpec((1, D), lambda i: (0, 0)),
        ],
        out_specs=pl.BlockSpec((BLK, D), lambda i: (i, 0)),
        out_shape=jax.ShapeDtypeStruct((N, D), jnp.float32),
    )(x, w, b.reshape(1, D))


def _fc_pre_body(x_ref, a_ref, wa_ref, wb_ref, b_ref, pre_ref, st_ref):
    pre = (jnp.dot(x_ref[...], wa_ref[...], preferred_element_type=jnp.float32)
           + jnp.dot(a_ref[...], wb_ref[...], preferred_element_type=jnp.float32)
           + b_ref[...])
    pre = jnp.maximum(pre, 0.0)
    pre_ref[...] = pre
    st_ref[0, 0:1, :] = jnp.sum(pre, axis=0, keepdims=True)
    st_ref[0, 1:2, :] = jnp.sum(pre * pre, axis=0, keepdims=True)


def _fc_pre(x, a, wa, wb, b):
    return pl.pallas_call(
        _fc_pre_body,
        grid=(NB,),
        in_specs=[
            pl.BlockSpec((BLK, D), lambda i: (i, 0)),
            pl.BlockSpec((BLK, D), lambda i: (i, 0)),
            pl.BlockSpec((D, D), lambda i: (0, 0)),
            pl.BlockSpec((D, D), lambda i: (0, 0)),
            pl.BlockSpec((1, D), lambda i: (0, 0)),
        ],
        out_specs=[
            pl.BlockSpec((BLK, D), lambda i: (i, 0)),
            pl.BlockSpec((1, 2, D), lambda i: (i, 0, 0)),
        ],
        out_shape=[
            jax.ShapeDtypeStruct((N, D), jnp.float32),
            jax.ShapeDtypeStruct((NB, 2, D), jnp.float32),
        ],
    )(x, a, wa, wb, b.reshape(1, D))


def _bn_body(pre_ref, st_ref, g_ref, be_ref, w2_ref, b2_ref, out1_ref, m2_ref):
    s = jnp.sum(st_ref[...], axis=0)
    mean = s[0:1, :] / N
    var = s[1:2, :] / N - mean * mean
    inv = g_ref[...] * lax.rsqrt(var + 1e-5)
    x = (pre_ref[...] - mean) * inv + be_ref[...]
    nrm = jnp.sqrt(jnp.sum(x * x, axis=1, keepdims=True))
    out1 = x / (nrm + 1e-6)
    out1_ref[...] = out1
    m2_ref[...] = jnp.maximum(
        jnp.dot(out1, w2_ref[...], preferred_element_type=jnp.float32)
        + b2_ref[...], 0.0)


def _bn_m2(pre, stats, gamma, beta, w2, b2):
    return pl.pallas_call(
        _bn_body,
        grid=(NB,),
        in_specs=[
            pl.BlockSpec((BLK, D), lambda i: (i, 0)),
            pl.BlockSpec((NB, 2, D), lambda i: (0, 0, 0)),
            pl.BlockSpec((1, D), lambda i: (0, 0)),
            pl.BlockSpec((1, D), lambda i: (0, 0)),
            pl.BlockSpec((D, D), lambda i: (0, 0)),
            pl.BlockSpec((1, D), lambda i: (0, 0)),
        ],
        out_specs=[
            pl.BlockSpec((BLK, D), lambda i: (i, 0)),
            pl.BlockSpec((BLK, D), lambda i: (i, 0)),
        ],
        out_shape=[
            jax.ShapeDtypeStruct((N, D), jnp.float32),
            jax.ShapeDtypeStruct((N, D), jnp.float32),
        ],
    )(pre, stats, gamma.reshape(1, D), beta.reshape(1, D), w2, b2.reshape(1, D))


def _fc_final_body(x_ref, a_ref, wa_ref, wb_ref, b_ref, o_ref):
    o_ref[...] = (
        jnp.dot(x_ref[...], wa_ref[...], preferred_element_type=jnp.float32)
        + jnp.dot(a_ref[...], wb_ref[...], preferred_element_type=jnp.float32)
        + b_ref[...])


def _fc_final(x, a, wa, wb, b):
    return pl.pallas_call(
        _fc_final_body,
        grid=(NB,),
        in_specs=[
            pl.BlockSpec((BLK, D), lambda i: (i, 0)),
            pl.BlockSpec((BLK, D), lambda i: (i, 0)),
            pl.BlockSpec((D, D), lambda i: (0, 0)),
            pl.BlockSpec((D, D), lambda i: (0, 0)),
            pl.BlockSpec((1, D), lambda i: (0, 0)),
        ],
        out_specs=pl.BlockSpec((BLK, D), lambda i: (i, 0)),
        out_shape=jax.ShapeDtypeStruct((N, D), jnp.float32),
    )(x, a, wa, wb, b.reshape(1, D))


def kernel(features, edge_index, W_agg1, b_agg1, W_fc1, b_fc1, gamma, beta,
           W_agg2, b_agg2, W_fc2, b_fc2):
    src = edge_index[0]
    dst = edge_index[1]

    srcl, offl, cnts = _compact(src, dst)
    m1 = _mm_relu(features, W_agg1, b_agg1)
    agg1 = _segmax(m1, srcl, offl, cnts)[:N]
    pre, stats = _fc_pre(features, agg1, W_fc1[:D], W_fc1[D:], b_fc1)
    out1, m2 = _bn_m2(pre, stats, gamma, beta, W_agg2, b_agg2)
    agg2 = _segmax(m2, srcl, offl, cnts)[:N]
    return _fc_final(out1, agg2, W_fc2[:D], W_fc2[D:], b_fc2)


# speedup vs baseline: 1.9564x; 1.1642x over previous
import functools

import jax
import jax.numpy as jnp
from jax import lax
from jax.experimental import pallas as pl
from jax.experimental.pallas import tpu as pltpu
from jax.experimental.pallas import tpu_sc as plsc

N = 10000
E = 320000
D = 128

NC = 2
NS = 16
NW = NC * NS
L = 16

OWN = 320
NPAD = OWN * NW
G = 128
SUBCAP = 896
CAP = SUBCAP * L
CHUNK = 6400
NCH = E // CHUNK

_mesh = plsc.VectorSubcoreMesh(core_axis_name="c", subcore_axis_name="s")


def _wid():
    return lax.axis_index("c") * NS + lax.axis_index("s")


@functools.partial(
    pl.kernel,
    out_type=(
        jax.ShapeDtypeStruct((NW, CAP), jnp.int32),
        jax.ShapeDtypeStruct((NW, CAP), jnp.int32),
        jax.ShapeDtypeStruct((NW, L), jnp.int32),
    ),
    mesh=_mesh,
    compiler_params=pltpu.CompilerParams(needs_layout_passes=False),
    scratch_types=[
        pltpu.VMEM((2, CHUNK), jnp.int32),
        pltpu.VMEM((2, CHUNK), jnp.int32),
        pltpu.VMEM((CAP,), jnp.int32),
        pltpu.VMEM((CAP,), jnp.int32),
        pltpu.VMEM((CAP,), jnp.int32),
        pltpu.VMEM((CAP,), jnp.int32),
        pltpu.VMEM((L,), jnp.int32),
        pltpu.SemaphoreType.DMA,
        pltpu.SemaphoreType.DMA,
    ],
)
def _compact(src_hbm, dst_hbm, srcl_hbm, offl_hbm, cnt_hbm,
             srcc_v, dstc_v, srcs_v, offs_v, srcl_v, offl_v, cnt_v,
             sem0, sem1):
    wid = _wid()
    lo = wid * OWN
    hi = lo + OWN
    lane_base = lax.iota(jnp.int32, L) * SUBCAP
    sems = (sem0, sem1)

    def start_chunk(ci, b):
        base = ci * CHUNK
        pltpu.make_async_copy(src_hbm.at[pl.ds(base, CHUNK)],
                              srcc_v.at[b], sems[b]).start()
        pltpu.make_async_copy(dst_hbm.at[pl.ds(base, CHUNK)],
                              dstc_v.at[b], sems[b]).start()

    def wait_chunk(b):
        pltpu.make_async_copy(src_hbm.at[pl.ds(0, CHUNK)],
                              srcc_v.at[b], sems[b]).wait()
        pltpu.make_async_copy(dst_hbm.at[pl.ds(0, CHUNK)],
                              dstc_v.at[b], sems[b]).wait()

    start_chunk(0, 0)

    def outer(gg, ptrs):
        for b in range(2):
            ci = gg * 2 + b

            @pl.when(ci + 1 < NCH)
            def _():
                start_chunk(ci + 1, 1 - b)

            wait_chunk(b)

            UNROLL = 4

            def vec_body(i, p):
                base_i = i * (UNROLL * L)
                dvs = [dstc_v[b, pl.ds(base_i + u * L, L)]
                       for u in range(UNROLL)]
                svs = [srcc_v[b, pl.ds(base_i + u * L, L)]
                       for u in range(UNROLL)]
                os_ = [dv - lo for dv in dvs]
                ms = [o.astype(jnp.uint32) < jnp.uint32(OWN) for o in os_]
                for u in range(UNROLL):
                    pos = lane_base + p
                    plsc.store_scatter(srcs_v, [pos], svs[u], mask=ms[u])
                    plsc.store_scatter(offs_v, [pos], os_[u], mask=ms[u])
                    p = p + ms[u].astype(jnp.int32)
                return jnp.minimum(p, SUBCAP - L)

            ptrs = lax.fori_loop(0, CHUNK // (UNROLL * L), vec_body, ptrs)
        return ptrs

    ptrs = lax.fori_loop(0, NCH // 2, outer, jnp.zeros((L,), jnp.int32))

    off = jnp.int32(0)
    for l in range(L):
        c_l = ptrs[l]
        src_base = l * SUBCAP

        def cp(i, _, off=off, src_base=src_base):
            srcl_v[pl.ds(off + i * L, L)] = srcs_v[pl.ds(src_base + i * L, L)]
            offl_v[pl.ds(off + i * L, L)] = offs_v[pl.ds(src_base + i * L, L)]
            return 0

        lax.fori_loop(0, (c_l + (L - 1)) // L, cp, 0)
        off = off + c_l

    pad_s = jnp.zeros((L,), jnp.int32)
    pad_o = jnp.full((L,), OWN, jnp.int32)

    def pad_body(j, _):
        srcl_v[pl.ds(off + j * L, L)] = pad_s
        offl_v[pl.ds(off + j * L, L)] = pad_o
        return 0

    lax.fori_loop(0, G // L, pad_body, 0)

    cnt_v[...] = jnp.full((L,), off, jnp.int32)
    pltpu.sync_copy(srcl_v, srcl_hbm.at[wid])
    pltpu.sync_copy(offl_v, offl_hbm.at[wid])
    pltpu.sync_copy(cnt_v, cnt_hbm.at[wid])


@functools.partial(
    pl.kernel,
    out_type=jax.ShapeDtypeStruct((NPAD, D), jnp.float32),
    mesh=_mesh,
    scratch_types=[
        pltpu.VMEM((CAP,), jnp.int32),
        pltpu.VMEM((CAP,), jnp.int32),
        pltpu.VMEM((L,), jnp.int32),
        pltpu.VMEM((OWN + 8, D), jnp.float32),
        pltpu.VMEM((2, G, D), jnp.float32),
        pltpu.SemaphoreType.DMA,
        pltpu.SemaphoreType.DMA,
    ],
)
def _segmax(m_hbm, srcl_hbm, offl_hbm, cnt_hbm, agg_hbm,
            srcl_v, offl_v, cnt_v, acc_v, rows_v, sem0, sem1):
    wid = _wid()
    pltpu.sync_copy(srcl_hbm.at[wid], srcl_v)
    pltpu.sync_copy(offl_hbm.at[wid], offl_v)
    pltpu.sync_copy(cnt_hbm.at[wid], cnt_v)

    zero = jnp.zeros((L,), jnp.float32)

    def zrow(r, _):
        for c in range(D // L):
            acc_v[r, pl.ds(c * L, L)] = zero
        return 0

    lax.fori_loop(0, OWN, zrow, 0)

    cnt = cnt_v[...][0]
    ngroups = (cnt + (G - 1)) // G
    sems = (sem0, sem1)

    def start_gather(g, b):
        pltpu.make_async_copy(m_hbm.at[srcl_v.at[pl.ds(g * G, G)]],
                              rows_v.at[b], sems[b]).start()

    def wait_gather(b):
        pltpu.make_async_copy(m_hbm.at[srcl_v.at[pl.ds(0, G)]],
                              rows_v.at[b], sems[b]).wait()

    @pl.when(ngroups > 0)
    def _():
        start_gather(0, 0)

    NC_ = D // L

    def outer(gg, _):
        for b in range(2):
            g = gg * 2 + b

            @pl.when(g < ngroups)
            def _():
                @pl.when(g + 1 < ngroups)
                def _():
                    start_gather(g + 1, 1 - b)

                wait_gather(b)

                def blk_body(k, _):
                    ovec = offl_v[pl.ds(g * G + k * L, L)]
                    dsts = [ovec[j] for j in range(L)]
                    for j in range(L):
                        d = dsts[j]
                        e = k * L + j
                        rvals = [rows_v[b, e, pl.ds(c * L, L)]
                                 for c in range(NC_)]
                        avals = [acc_v[d, pl.ds(c * L, L)]
                                 for c in range(NC_)]
                        for c in range(NC_):
                            acc_v[d, pl.ds(c * L, L)] = jnp.maximum(
                                avals[c], rvals[c])
                    return 0

                lax.fori_loop(0, G // L, blk_body, 0)
        return 0

    lax.fori_loop(0, (ngroups + 1) // 2, outer, 0)
    pltpu.sync_copy(acc_v.at[pl.ds(0, OWN)], agg_hbm.at[pl.ds(wid * OWN, OWN)])


BLK = 2000
NB = N // BLK


def _mm_relu_body(x_ref, w_ref, b_ref, o_ref):
    o_ref[...] = jnp.maximum(
        jnp.dot(x_ref[...], w_ref[...], preferred_element_type=jnp.float32)
        + b_ref[...], 0.0)


def _mm_relu(x, w, b):
    return pl.pallas_call(
        _mm_relu_body,
        grid=(NB,),
        in_specs=[
            pl.BlockSpec((BLK, D), lambda i: (i, 0)),
            pl.BlockSpec((D, D), lambda i: (0, 0)),
            pl.BlockSpec((1, D), lambda i: (0, 0)),
        ],
        out_specs=pl.BlockSpec((BLK, D), lambda i: (i, 0)),
        out_shape=jax.ShapeDtypeStruct((N, D), jnp.float32),
    )(x, w, b.reshape(1, D))


def _fc_pre_body(x_ref, a_ref, wa_ref, wb_ref, b_ref, pre_ref, st_ref):
    pre = (jnp.dot(x_ref[...], wa_ref[...], preferred_element_type=jnp.float32)
           + jnp.dot(a_ref[...], wb_ref[...], preferred_element_type=jnp.float32)
           + b_ref[...])
    pre = jnp.maximum(pre, 0.0)
    pre_ref[...] = pre
    st_ref[0, 0:1, :] = jnp.sum(pre, axis=0, keepdims=True)
    st_ref[0, 1:2, :] = jnp.sum(pre * pre, axis=0, keepdims=True)


def _fc_pre(x, a, wa, wb, b):
    return pl.pallas_call(
        _fc_pre_body,
        grid=(NB,),
        in_specs=[
            pl.BlockSpec((BLK, D), lambda i: (i, 0)),
            pl.BlockSpec((BLK, D), lambda i: (i, 0)),
            pl.BlockSpec((D, D), lambda i: (0, 0)),
            pl.BlockSpec((D, D), lambda i: (0, 0)),
            pl.BlockSpec((1, D), lambda i: (0, 0)),
        ],
        out_specs=[
            pl.BlockSpec((BLK, D), lambda i: (i, 0)),
            pl.BlockSpec((1, 2, D), lambda i: (i, 0, 0)),
        ],
        out_shape=[
            jax.ShapeDtypeStruct((N, D), jnp.float32),
            jax.ShapeDtypeStruct((NB, 2, D), jnp.float32),
        ],
    )(x, a, wa, wb, b.reshape(1, D))


def _bn_body(pre_ref, st_ref, g_ref, be_ref, w2_ref, b2_ref, out1_ref, m2_ref):
    s = jnp.sum(st_ref[...], axis=0)
    mean = s[0:1, :] / N
    var = s[1:2, :] / N - mean * mean
    inv = g_ref[...] * lax.rsqrt(var + 1e-5)
    x = (pre_ref[...] - mean) * inv + be_ref[...]
    nrm = jnp.sqrt(jnp.sum(x * x, axis=1, keepdims=True))
    out1 = x / (nrm + 1e-6)
    out1_ref[...] = out1
    m2_ref[...] = jnp.maximum(
        jnp.dot(out1, w2_ref[...], preferred_element_type=jnp.float32)
        + b2_ref[...], 0.0)


def _bn_m2(pre, stats, gamma, beta, w2, b2):
    return pl.pallas_call(
        _bn_body,
        grid=(NB,),
        in_specs=[
            pl.BlockSpec((BLK, D), lambda i: (i, 0)),
            pl.BlockSpec((NB, 2, D), lambda i: (0, 0, 0)),
            pl.BlockSpec((1, D), lambda i: (0, 0)),
            pl.BlockSpec((1, D), lambda i: (0, 0)),
            pl.BlockSpec((D, D), lambda i: (0, 0)),
            pl.BlockSpec((1, D), lambda i: (0, 0)),
        ],
        out_specs=[
            pl.BlockSpec((BLK, D), lambda i: (i, 0)),
            pl.BlockSpec((BLK, D), lambda i: (i, 0)),
        ],
        out_shape=[
            jax.ShapeDtypeStruct((N, D), jnp.float32),
            jax.ShapeDtypeStruct((N, D), jnp.float32),
        ],
    )(pre, stats, gamma.reshape(1, D), beta.reshape(1, D), w2, b2.reshape(1, D))


def _fc_final_body(x_ref, a_ref, wa_ref, wb_ref, b_ref, o_ref):
    o_ref[...] = (
        jnp.dot(x_ref[...], wa_ref[...], preferred_element_type=jnp.float32)
        + jnp.dot(a_ref[...], wb_ref[...], preferred_element_type=jnp.float32)
        + b_ref[...])


def _fc_final(x, a, wa, wb, b):
    return pl.pallas_call(
        _fc_final_body,
        grid=(NB,),
        in_specs=[
            pl.BlockSpec((BLK, D), lambda i: (i, 0)),
            pl.BlockSpec((BLK, D), lambda i: (i, 0)),
            pl.BlockSpec((D, D), lambda i: (0, 0)),
            pl.BlockSpec((D, D), lambda i: (0, 0)),
            pl.BlockSpec((1, D), lambda i: (0, 0)),
        ],
        out_specs=pl.BlockSpec((BLK, D), lambda i: (i, 0)),
        out_shape=jax.ShapeDtypeStruct((N, D), jnp.float32),
    )(x, a, wa, wb, b.reshape(1, D))


def kernel(features, edge_index, W_agg1, b_agg1, W_fc1, b_fc1, gamma, beta,
           W_agg2, b_agg2, W_fc2, b_fc2):
    src = edge_index[0]
    dst = edge_index[1]

    srcl, offl, cnts = _compact(src, dst)
    m1 = _mm_relu(features, W_agg1, b_agg1)
    agg1 = _segmax(m1, srcl, offl, cnts)[:N]
    pre, stats = _fc_pre(features, agg1, W_fc1[:D], W_fc1[D:], b_fc1)
    out1, m2 = _bn_m2(pre, stats, gamma, beta, W_agg2, b_agg2)
    agg2 = _segmax(m2, srcl, offl, cnts)[:N]
    return _fc_final(out1, agg2, W_fc2[:D], W_fc2[D:], b_fc2)


# speedup vs baseline: 2.1523x; 1.1001x over previous
import functools

import jax
import jax.numpy as jnp
from jax import lax
from jax.experimental import pallas as pl
from jax.experimental.pallas import tpu as pltpu
from jax.experimental.pallas import tpu_sc as plsc

N = 10000
E = 320000
D = 128

NC = 2
NS = 16
NW = NC * NS
L = 16

OWN = 320
NPAD = OWN * NW
G = 128
SUBCAP = 896
CAP = SUBCAP * L
CHUNK = 6400
NCH = E // CHUNK

_mesh = plsc.VectorSubcoreMesh(core_axis_name="c", subcore_axis_name="s")


def _wid():
    return lax.axis_index("c") * NS + lax.axis_index("s")


@functools.partial(
    pl.kernel,
    out_type=(
        jax.ShapeDtypeStruct((NW, CAP), jnp.int32),
        jax.ShapeDtypeStruct((NW, CAP), jnp.int32),
        jax.ShapeDtypeStruct((NW, L), jnp.int32),
    ),
    mesh=_mesh,
    compiler_params=pltpu.CompilerParams(needs_layout_passes=False),
    scratch_types=[
        pltpu.VMEM((2, CHUNK), jnp.int32),
        pltpu.VMEM((2, CHUNK), jnp.int32),
        pltpu.VMEM((CAP,), jnp.int32),
        pltpu.VMEM((CAP,), jnp.int32),
        pltpu.VMEM((CAP,), jnp.int32),
        pltpu.VMEM((CAP,), jnp.int32),
        pltpu.VMEM((L,), jnp.int32),
        pltpu.SemaphoreType.DMA,
        pltpu.SemaphoreType.DMA,
    ],
)
def _compact(src_hbm, dst_hbm, srcl_hbm, offl_hbm, cnt_hbm,
             srcc_v, dstc_v, srcs_v, offs_v, srcl_v, offl_v, cnt_v,
             sem0, sem1):
    wid = _wid()
    lo = wid * OWN
    hi = lo + OWN
    lane_base = lax.iota(jnp.int32, L) * SUBCAP
    sems = (sem0, sem1)

    def start_chunk(ci, b):
        base = ci * CHUNK
        pltpu.make_async_copy(src_hbm.at[pl.ds(base, CHUNK)],
                              srcc_v.at[b], sems[b]).start()
        pltpu.make_async_copy(dst_hbm.at[pl.ds(base, CHUNK)],
                              dstc_v.at[b], sems[b]).start()

    def wait_chunk(b):
        pltpu.make_async_copy(src_hbm.at[pl.ds(0, CHUNK)],
                              srcc_v.at[b], sems[b]).wait()
        pltpu.make_async_copy(dst_hbm.at[pl.ds(0, CHUNK)],
                              dstc_v.at[b], sems[b]).wait()

    start_chunk(0, 0)

    def outer(gg, ptrs):
        for b in range(2):
            ci = gg * 2 + b

            @pl.when(ci + 1 < NCH)
            def _():
                start_chunk(ci + 1, 1 - b)

            wait_chunk(b)

            UNROLL = 4

            def vec_body(i, p):
                base_i = i * (UNROLL * L)
                dvs = [dstc_v[b, pl.ds(base_i + u * L, L)]
                       for u in range(UNROLL)]
                svs = [srcc_v[b, pl.ds(base_i + u * L, L)]
                       for u in range(UNROLL)]
                os_ = [dv - lo for dv in dvs]
                ms = [o.astype(jnp.uint32) < jnp.uint32(OWN) for o in os_]
                for u in range(UNROLL):
                    pos = lane_base + p
                    plsc.store_scatter(srcs_v, [pos], svs[u], mask=ms[u])
                    plsc.store_scatter(offs_v, [pos], os_[u], mask=ms[u])
                    p = p + ms[u].astype(jnp.int32)
                return jnp.minimum(p, SUBCAP - L)

            ptrs = lax.fori_loop(0, CHUNK // (UNROLL * L), vec_body, ptrs)
        return ptrs

    ptrs = lax.fori_loop(0, NCH // 2, outer, jnp.zeros((L,), jnp.int32))

    off = jnp.int32(0)
    for l in range(L):
        c_l = ptrs[l]
        src_base = l * SUBCAP

        def cp(i, _, off=off, src_base=src_base):
            srcl_v[pl.ds(off + i * L, L)] = srcs_v[pl.ds(src_base + i * L, L)]
            offl_v[pl.ds(off + i * L, L)] = offs_v[pl.ds(src_base + i * L, L)]
            return 0

        lax.fori_loop(0, (c_l + (L - 1)) // L, cp, 0)
        off = off + c_l

    pad_s = jnp.zeros((L,), jnp.int32)
    pad_o = jnp.full((L,), OWN, jnp.int32)

    def pad_body(j, _):
        srcl_v[pl.ds(off + j * L, L)] = pad_s
        offl_v[pl.ds(off + j * L, L)] = pad_o
        return 0

    lax.fori_loop(0, G // L, pad_body, 0)

    cnt_v[...] = jnp.full((L,), off, jnp.int32)
    pltpu.sync_copy(srcl_v, srcl_hbm.at[wid])
    pltpu.sync_copy(offl_v, offl_hbm.at[wid])
    pltpu.sync_copy(cnt_v, cnt_hbm.at[wid])


@functools.partial(
    pl.kernel,
    out_type=jax.ShapeDtypeStruct((NPAD, D // 2), jnp.int32),
    mesh=_mesh,
    compiler_params=pltpu.CompilerParams(needs_layout_passes=False,
                                         use_tc_tiling_on_sc=False),
    scratch_types=[
        pltpu.VMEM((CAP,), jnp.int32),
        pltpu.VMEM((CAP,), jnp.int32),
        pltpu.VMEM((L,), jnp.int32),
        pltpu.VMEM((OWN + 8, D // 2), jnp.int32),
        pltpu.VMEM((2, G, D // 2), jnp.int32),
        pltpu.SemaphoreType.DMA,
        pltpu.SemaphoreType.DMA,
    ],
)
def _segmax(m_hbm, srcl_hbm, offl_hbm, cnt_hbm, agg_hbm,
            srcl_v, offl_v, cnt_v, acc_v, rows_v, sem0, sem1):
    wid = _wid()
    pltpu.sync_copy(srcl_hbm.at[wid], srcl_v)
    pltpu.sync_copy(offl_hbm.at[wid], offl_v)
    pltpu.sync_copy(cnt_hbm.at[wid], cnt_v)

    zero = jnp.zeros((L,), jnp.int32)

    def zrow(r, _):
        for c in range(D // (2 * L)):
            acc_v[r, pl.ds(c * L, L)] = zero
        return 0

    lax.fori_loop(0, OWN, zrow, 0)

    cnt = cnt_v[...][0]
    ngroups = (cnt + (G - 1)) // G
    sems = (sem0, sem1)

    def start_gather(g, b):
        pltpu.make_async_copy(m_hbm.at[srcl_v.at[pl.ds(g * G, G)]],
                              rows_v.at[b], sems[b]).start()

    def wait_gather(b):
        pltpu.make_async_copy(m_hbm.at[srcl_v.at[pl.ds(0, G)]],
                              rows_v.at[b], sems[b]).wait()

    @pl.when(ngroups > 0)
    def _():
        start_gather(0, 0)

    NC_ = D // (2 * L)

    def outer(gg, _):
        for b in range(2):
            g = gg * 2 + b

            @pl.when(g < ngroups)
            def _():
                @pl.when(g + 1 < ngroups)
                def _():
                    start_gather(g + 1, 1 - b)

                wait_gather(b)

                def blk_body(k, _):
                    ovec = offl_v[pl.ds(g * G + k * L, L)]
                    dsts = [ovec[j] for j in range(L)]
                    for j in range(L):
                        d = dsts[j]
                        e = k * L + j
                        rvals = [rows_v[b, e, pl.ds(c * L, L)]
                                 for c in range(NC_)]
                        avals = [acc_v[d, pl.ds(c * L, L)]
                                 for c in range(NC_)]
                        for c in range(NC_):
                            mx = jnp.maximum(
                                plsc.bitcast(avals[c], jnp.bfloat16),
                                plsc.bitcast(rvals[c], jnp.bfloat16))
                            acc_v[d, pl.ds(c * L, L)] = plsc.bitcast(
                                mx, jnp.int32)
                    return 0

                lax.fori_loop(0, G // L, blk_body, 0)
        return 0

    lax.fori_loop(0, (ngroups + 1) // 2, outer, 0)
    pltpu.sync_copy(acc_v.at[pl.ds(0, OWN)], agg_hbm.at[pl.ds(wid * OWN, OWN)])


BLK = 2000
NB = N // BLK


def _mm_relu_body(x_ref, w_ref, b_ref, o_ref):
    o_ref[...] = jnp.maximum(
        jnp.dot(x_ref[...], w_ref[...], preferred_element_type=jnp.float32)
        + b_ref[...], 0.0).astype(jnp.bfloat16)


def _mm_relu(x, w, b):
    return pl.pallas_call(
        _mm_relu_body,
        grid=(NB,),
        in_specs=[
            pl.BlockSpec((BLK, D), lambda i: (i, 0)),
            pl.BlockSpec((D, D), lambda i: (0, 0)),
            pl.BlockSpec((1, D), lambda i: (0, 0)),
        ],
        out_specs=pl.BlockSpec((BLK, D), lambda i: (i, 0)),
        out_shape=jax.ShapeDtypeStruct((N, D), jnp.bfloat16),
    )(x, w, b.reshape(1, D))


def _fc_pre_body(x_ref, a_ref, wa_ref, wb_ref, b_ref, pre_ref, st_ref):
    pre = (jnp.dot(x_ref[...], wa_ref[...], preferred_element_type=jnp.float32)
           + jnp.dot(a_ref[...].astype(jnp.float32), wb_ref[...],
                     preferred_element_type=jnp.float32)
           + b_ref[...])
    pre = jnp.maximum(pre, 0.0)
    pre_ref[...] = pre
    st_ref[0, 0:1, :] = jnp.sum(pre, axis=0, keepdims=True)
    st_ref[0, 1:2, :] = jnp.sum(pre * pre, axis=0, keepdims=True)


def _fc_pre(x, a, wa, wb, b):
    return pl.pallas_call(
        _fc_pre_body,
        grid=(NB,),
        in_specs=[
            pl.BlockSpec((BLK, D), lambda i: (i, 0)),
            pl.BlockSpec((BLK, D), lambda i: (i, 0)),
            pl.BlockSpec((D, D), lambda i: (0, 0)),
            pl.BlockSpec((D, D), lambda i: (0, 0)),
            pl.BlockSpec((1, D), lambda i: (0, 0)),
        ],
        out_specs=[
            pl.BlockSpec((BLK, D), lambda i: (i, 0)),
            pl.BlockSpec((1, 2, D), lambda i: (i, 0, 0)),
        ],
        out_shape=[
            jax.ShapeDtypeStruct((N, D), jnp.float32),
            jax.ShapeDtypeStruct((NB, 2, D), jnp.float32),
        ],
    )(x, a, wa, wb, b.reshape(1, D))


def _bn_body(pre_ref, st_ref, g_ref, be_ref, w2_ref, b2_ref, out1_ref, m2_ref):
    s = jnp.sum(st_ref[...], axis=0)
    mean = s[0:1, :] / N
    var = s[1:2, :] / N - mean * mean
    inv = g_ref[...] * lax.rsqrt(var + 1e-5)
    x = (pre_ref[...] - mean) * inv + be_ref[...]
    nrm = jnp.sqrt(jnp.sum(x * x, axis=1, keepdims=True))
    out1 = x / (nrm + 1e-6)
    out1_ref[...] = out1
    m2_ref[...] = jnp.maximum(
        jnp.dot(out1, w2_ref[...], preferred_element_type=jnp.float32)
        + b2_ref[...], 0.0).astype(jnp.bfloat16)


def _bn_m2(pre, stats, gamma, beta, w2, b2):
    return pl.pallas_call(
        _bn_body,
        grid=(NB,),
        in_specs=[
            pl.BlockSpec((BLK, D), lambda i: (i, 0)),
            pl.BlockSpec((NB, 2, D), lambda i: (0, 0, 0)),
            pl.BlockSpec((1, D), lambda i: (0, 0)),
            pl.BlockSpec((1, D), lambda i: (0, 0)),
            pl.BlockSpec((D, D), lambda i: (0, 0)),
            pl.BlockSpec((1, D), lambda i: (0, 0)),
        ],
        out_specs=[
            pl.BlockSpec((BLK, D), lambda i: (i, 0)),
            pl.BlockSpec((BLK, D), lambda i: (i, 0)),
        ],
        out_shape=[
            jax.ShapeDtypeStruct((N, D), jnp.float32),
            jax.ShapeDtypeStruct((N, D), jnp.bfloat16),
        ],
    )(pre, stats, gamma.reshape(1, D), beta.reshape(1, D), w2, b2.reshape(1, D))


def _fc_final_body(x_ref, a_ref, wa_ref, wb_ref, b_ref, o_ref):
    o_ref[...] = (
        jnp.dot(x_ref[...], wa_ref[...], preferred_element_type=jnp.float32)
        + jnp.dot(a_ref[...].astype(jnp.float32), wb_ref[...],
                  preferred_element_type=jnp.float32)
        + b_ref[...])


def _fc_final(x, a, wa, wb, b):
    return pl.pallas_call(
        _fc_final_body,
        grid=(NB,),
        in_specs=[
            pl.BlockSpec((BLK, D), lambda i: (i, 0)),
            pl.BlockSpec((BLK, D), lambda i: (i, 0)),
            pl.BlockSpec((D, D), lambda i: (0, 0)),
            pl.BlockSpec((D, D), lambda i: (0, 0)),
            pl.BlockSpec((1, D), lambda i: (0, 0)),
        ],
        out_specs=pl.BlockSpec((BLK, D), lambda i: (i, 0)),
        out_shape=jax.ShapeDtypeStruct((N, D), jnp.float32),
    )(x, a, wa, wb, b.reshape(1, D))


def kernel(features, edge_index, W_agg1, b_agg1, W_fc1, b_fc1, gamma, beta,
           W_agg2, b_agg2, W_fc2, b_fc2):
    src = edge_index[0]
    dst = edge_index[1]

    srcl, offl, cnts = _compact(src, dst)

    def pack(m_bf16):
        return lax.bitcast_convert_type(
            m_bf16.reshape(N, D // 2, 2), jnp.int32)

    def unpack(agg_i32):
        return lax.bitcast_convert_type(
            agg_i32, jnp.bfloat16).reshape(NPAD, D)[:N]

    m1 = _mm_relu(features, W_agg1, b_agg1)
    agg1 = unpack(_segmax(pack(m1), srcl, offl, cnts))
    pre, stats = _fc_pre(features, agg1, W_fc1[:D], W_fc1[D:], b_fc1)
    out1, m2 = _bn_m2(pre, stats, gamma, beta, W_agg2, b_agg2)
    agg2 = unpack(_segmax(pack(m2), srcl, offl, cnts))
    return _fc_final(out1, agg2, W_fc2[:D], W_fc2[D:], b_fc2)
